# baseline (speedup 1.0000x reference)
import functools
import math

import jax
import jax.numpy as jnp
from jax.experimental import pallas as pl
from jax.experimental.pallas import tpu as pltpu


def _linear_bf16_kernel(x_ref, w_ref, b_ref, o_ref, *, tn: int):
    j = pl.program_id(1)
    acc = jnp.dot(
        x_ref[...],
        w_ref[...],
        preferred_element_type=jnp.float32,
    )
    b = b_ref[:, pl.ds(j * tn, tn)]
    o_ref[...] = (acc + b).astype(o_ref.dtype)


def _ceil_to(a: int, b: int) -> int:
    return -(-a // b) * b


def kernel(x, weight, bias):
    out_dtype = x.dtype

    if x.ndim == 3 and x.shape[1] == 1:
        x = jnp.squeeze(x, axis=1)

    K, N = weight.shape
    lead_shape = x.shape[:-1]
    M = int(math.prod(lead_shape)) if lead_shape else 1

    x2d = x.reshape(M, K)
    w = weight
    b2d = bias.astype(jnp.float32).reshape(1, N)

    tm = min(1024, _ceil_to(M, 8))
    tn = min(512, _ceil_to(N, 128))
    Mp, Np, Kp = _ceil_to(M, tm), _ceil_to(N, tn), _ceil_to(K, 128)
    if (Mp, Kp) != (M, K):
        x2d = jnp.pad(x2d, ((0, Mp - M), (0, Kp - K)))
    if (Kp, Np) != (K, N):
        w = jnp.pad(w, ((0, Kp - K), (0, Np - N)))
        b2d = jnp.pad(b2d, ((0, 0), (0, Np - N)))

    grid = (Mp // tm, Np // tn)

    cost = pl.CostEstimate(
        flops=2 * Mp * Kp * Np,
        transcendentals=0,
        bytes_accessed=2 * (Np // tn) * Mp * Kp + 2 * (Mp // tm) * Kp * Np
        + 4 * Mp * Np,
    )

    out = pl.pallas_call(
        functools.partial(_linear_bf16_kernel, tn=tn),
        out_shape=jax.ShapeDtypeStruct((Mp, Np), out_dtype),
        grid=grid,
        in_specs=[
            pl.BlockSpec((tm, Kp), lambda i, j: (i, 0)),
            pl.BlockSpec((Kp, tn), lambda i, j: (0, j)),
            pl.BlockSpec((1, Np), lambda i, j: (0, 0)),
        ],
        out_specs=pl.BlockSpec((tm, tn), lambda i, j: (i, j)),
        compiler_params=pltpu.CompilerParams(
            dimension_semantics=("parallel", "parallel"),
        ),
        cost_estimate=cost,
    )(x2d, w, b2d)

    if (Mp, Np) != (M, N):
        out = out[:M, :N]
    return out.reshape(*lead_shape, N)


# speedup vs baseline: 1.0086x; 1.0086x over previous
import math

import jax
import jax.numpy as jnp
from jax.experimental import pallas as pl
from jax.experimental.pallas import tpu as pltpu


def _linear_bf16_kernel(x_ref, w_ref, b_ref, o_ref):
    acc = jnp.dot(
        x_ref[...],
        w_ref[...],
        preferred_element_type=jnp.float32,
    )
    o_ref[...] = (acc + b_ref[...]).astype(o_ref.dtype)


def _ceil_to(a: int, b: int) -> int:
    return -(-a // b) * b


def kernel(x, weight, bias):
    out_dtype = x.dtype

    if x.ndim == 3 and x.shape[1] == 1:
        x = jnp.squeeze(x, axis=1)

    K, N = weight.shape
    lead_shape = x.shape[:-1]
    M = int(math.prod(lead_shape)) if lead_shape else 1

    x2d = x.reshape(M, K)
    w = weight
    b2d = bias.astype(jnp.float32).reshape(1, N)

    tm = min(1024, _ceil_to(M, 8))
    tn = min(512, _ceil_to(N, 128))
    Mp, Np, Kp = _ceil_to(M, tm), _ceil_to(N, tn), _ceil_to(K, 128)
    if (Mp, Kp) != (M, K):
        x2d = jnp.pad(x2d, ((0, Mp - M), (0, Kp - K)))
    if (Kp, Np) != (K, N):
        w = jnp.pad(w, ((0, Kp - K), (0, Np - N)))
        b2d = jnp.pad(b2d, ((0, 0), (0, Np - N)))

    grid = (Mp // tm, Np // tn)

    cost = pl.CostEstimate(
        flops=2 * Mp * Kp * Np,
        transcendentals=0,
        bytes_accessed=2 * (Np // tn) * Mp * Kp + 2 * (Mp // tm) * Kp * Np
        + 4 * Mp * Np,
    )

    out = pl.pallas_call(
        _linear_bf16_kernel,
        out_shape=jax.ShapeDtypeStruct((Mp, Np), out_dtype),
        grid=grid,
        in_specs=[
            pl.BlockSpec((tm, Kp), lambda i, j: (i, 0)),
            pl.BlockSpec((Kp, tn), lambda i, j: (0, j)),
            pl.BlockSpec((1, tn), lambda i, j: (0, j)),
        ],
        out_specs=pl.BlockSpec((tm, tn), lambda i, j: (i, j)),
        compiler_params=pltpu.CompilerParams(
            dimension_semantics=("parallel", "parallel"),
        ),
        cost_estimate=cost,
    )(x2d, w, b2d)

    if (Mp, Np) != (M, N):
        out = out[:M, :N]
    return out.reshape(*lead_shape, N)
